# Initial kernel scaffold; baseline (speedup 1.0000x reference)
import jax
import jax.numpy as jnp
from jax.experimental import pallas as pl


def kernel(means, node_xyz_ref, node_quat_ref, node_xyz_t, node_quat_t, quats, skinning_weight, scales, opacities, shs, t):
    raise NotImplementedError("write your pallas kernel here")


# speedup vs baseline: 13.8383x; 13.8383x over previous
import functools

import jax
import jax.numpy as jnp
from jax import lax
from jax.experimental import pallas as pl
from jax.experimental.pallas import tpu as pltpu
from jax.experimental.pallas import tpu_sc as plsc

SIGMA = 0.5
DQ_EPS = 1e-6

_NC = 2
_NS = 16
_LANES = 16
_NW = _NC * _NS

_BLK_N = 512
_BLK_D = 4096


def _q2r_rows(q):
    n = jnp.sqrt(q[0:1] * q[0:1] + q[1:2] * q[1:2]
                 + q[2:3] * q[2:3] + q[3:4] * q[3:4]) + 1e-8
    w = q[0:1] / n
    x = q[1:2] / n
    y = q[2:3] / n
    z = q[3:4] / n
    return [
        1 - 2 * (y * y + z * z), 2 * (x * y - w * z), 2 * (x * z + w * y),
        2 * (x * y + w * z), 1 - 2 * (x * x + z * z), 2 * (y * z - w * x),
        2 * (x * z - w * y), 2 * (y * z + w * x), 1 - 2 * (x * x + y * y),
    ]


def _node_table_body(qr_ref, qt_ref, xr_ref, xt_ref, out_ref):
    rr = _q2r_rows(qr_ref[...])
    rt = _q2r_rows(qt_ref[...])
    xr = xr_ref[...]
    xt = xt_ref[...]
    a = []
    for i in range(3):
        for j in range(3):
            a.append(rt[3 * i + 0] * rr[3 * j + 0]
                     + rt[3 * i + 1] * rr[3 * j + 1]
                     + rt[3 * i + 2] * rr[3 * j + 2])
    b = [xt[i:i + 1]
         - (a[3 * i + 0] * xr[0:1] + a[3 * i + 1] * xr[1:2]
            + a[3 * i + 2] * xr[2:3])
         for i in range(3)]
    zero = a[0] * 0.0
    out_ref[...] = jnp.concatenate(a + b + [zero] * 4, axis=0)


def _node_table(qr_t, qt_t, xr_t, xt_t, m):
    return pl.pallas_call(
        _node_table_body,
        out_shape=jax.ShapeDtypeStruct((16, m), jnp.float32),
    )(qr_t, qt_t, xr_t, xt_t)


def _topk_body(means_ref, nodes_ref, sw_ref, idx_ref, w_ref):
    means_blk = means_ref[...]
    nodes_t = nodes_ref[...]
    bm = means_blk.shape[0]
    m = nodes_t.shape[1]
    mm = jnp.dot(means_blk, nodes_t, preferred_element_type=jnp.float32)
    rn = jnp.sum(means_blk * means_blk, axis=1, keepdims=True)
    cn = jnp.sum(nodes_t * nodes_t, axis=0, keepdims=True)
    d2 = rn + cn - 2.0 * mm
    col = lax.broadcasted_iota(jnp.int32, (bm, m), 1)
    big = jnp.int32(0x7FFFFFFF)
    vals, inds = [], []
    for k in range(8):
        mk = jnp.min(d2, axis=1, keepdims=True)
        ik = jnp.min(jnp.where(d2 == mk, col, big),
                     axis=1, keepdims=True)
        vals.append(mk)
        inds.append(ik)
        if k < 7:
            d2 = jnp.where(col == ik, jnp.float32(jnp.inf), d2)
    d2k = jnp.concatenate(vals, axis=1)
    idx = jnp.concatenate(inds, axis=1)
    w = jnp.exp(d2k * (-1.0 / (2.0 * SIGMA * SIGMA))) + sw_ref[...]
    w = jnp.maximum(w, 0.0)
    w = w / jnp.maximum(jnp.sum(w, axis=1, keepdims=True), DQ_EPS)
    idx_ref[...] = idx
    w_ref[...] = w


def _topk(means8, nodes8_t, sw, n, m, k):
    grid = n // _BLK_N
    return pl.pallas_call(
        _topk_body,
        grid=(grid,),
        in_specs=[
            pl.BlockSpec((_BLK_N, 8), lambda i: (i, 0)),
            pl.BlockSpec((8, m), lambda i: (0, 0)),
            pl.BlockSpec((_BLK_N, k), lambda i: (i, 0)),
        ],
        out_specs=[
            pl.BlockSpec((_BLK_N, k), lambda i: (i, 0)),
            pl.BlockSpec((_BLK_N, k), lambda i: (i, 0)),
        ],
        out_shape=[
            jax.ShapeDtypeStruct((n, k), jnp.int32),
            jax.ShapeDtypeStruct((n, k), jnp.float32),
        ],
    )(means8, nodes8_t, sw)


def _sc_blend(n, m, k):
    p = n // _NW
    groups = p // _LANES
    mesh = plsc.VectorSubcoreMesh(core_axis_name="c", subcore_axis_name="s",
                                  num_cores=_NC, num_subcores=_NS)

    @functools.partial(
        pl.kernel,
        out_type=jax.ShapeDtypeStruct((_NW, 12, p), jnp.float32),
        mesh=mesh,
        compiler_params=pltpu.CompilerParams(needs_layout_passes=False),
        scratch_types=[
            pltpu.VMEM((m * 16,), jnp.float32),
            pltpu.VMEM((k, p), jnp.int32),
            pltpu.VMEM((k, p), jnp.float32),
            pltpu.VMEM((12, p), jnp.float32),
        ],
    )
    def blend(ab_hbm, idx_hbm, w_hbm, out_hbm, ab_v, idx_v, w_v, out_v):
        wid = lax.axis_index("s") * _NC + lax.axis_index("c")
        pltpu.sync_copy(ab_hbm, ab_v)
        pltpu.sync_copy(idx_hbm.at[wid], idx_v)
        pltpu.sync_copy(w_hbm.at[wid], w_v)

        def body(g, carry):
            col = g * _LANES
            acc = [jnp.zeros((_LANES,), jnp.float32) for _ in range(12)]
            for kk in range(k):
                iv = idx_v[kk, pl.ds(col, _LANES)] * 16
                wv = w_v[kk, pl.ds(col, _LANES)]
                for c in range(12):
                    acc[c] = acc[c] + wv * plsc.load_gather(ab_v, [iv + c])
            for c in range(12):
                out_v[c, pl.ds(col, _LANES)] = acc[c]
            return carry

        lax.fori_loop(0, groups, body, 0)
        pltpu.sync_copy(out_v, out_hbm.at[wid])

    return blend


def _final_body(blend_ref, means_ref, quats_ref, mu_ref, fr_ref):
    f = [blend_ref[i:i + 1] for i in range(9)]
    b = [blend_ref[9 + i:10 + i] for i in range(3)]
    mx = [means_ref[i:i + 1] for i in range(3)]
    rq = _q2r_rows(quats_ref[...])
    mu = [f[3 * i + 0] * mx[0] + f[3 * i + 1] * mx[1]
          + f[3 * i + 2] * mx[2] + b[i] for i in range(3)]
    fr = []
    for i in range(3):
        for j in range(3):
            fr.append(f[3 * i + 0] * rq[0 + j] + f[3 * i + 1] * rq[3 + j]
                      + f[3 * i + 2] * rq[6 + j])
    mu_ref[...] = jnp.concatenate(mu, axis=0)
    fr_ref[...] = jnp.concatenate(fr, axis=0)


def _finalize(blend_t, means_t, quats_t, n):
    grid = n // _BLK_D
    return pl.pallas_call(
        _final_body,
        grid=(grid,),
        in_specs=[
            pl.BlockSpec((12, _BLK_D), lambda i: (0, i)),
            pl.BlockSpec((3, _BLK_D), lambda i: (0, i)),
            pl.BlockSpec((4, _BLK_D), lambda i: (0, i)),
        ],
        out_specs=[
            pl.BlockSpec((3, _BLK_D), lambda i: (0, i)),
            pl.BlockSpec((9, _BLK_D), lambda i: (0, i)),
        ],
        out_shape=[
            jax.ShapeDtypeStruct((3, n), jnp.float32),
            jax.ShapeDtypeStruct((9, n), jnp.float32),
        ],
    )(blend_t, means_t, quats_t)


def kernel(means, node_xyz_ref, node_quat_ref, node_xyz_t, node_quat_t,
           quats, skinning_weight, scales, opacities, shs, t):
    n = means.shape[0]
    m = node_xyz_ref.shape[0]
    k = skinning_weight.shape[1]

    ab16 = _node_table(node_quat_ref.T, node_quat_t.T,
                       node_xyz_ref.T, node_xyz_t.T, m)
    ab_flat = ab16.T.reshape(m * 16)

    means8 = jnp.concatenate(
        [means, jnp.zeros((n, 5), jnp.float32)], axis=1)
    nodes8_t = jnp.concatenate(
        [node_xyz_ref.T, jnp.zeros((5, m), jnp.float32)], axis=0)
    idx, w = _topk(means8, nodes8_t, skinning_weight, n, m, k)

    p = n // _NW
    idx3 = idx.reshape(_NW, p, k).transpose(0, 2, 1)
    w3 = w.reshape(_NW, p, k).transpose(0, 2, 1)
    out3 = _sc_blend(n, m, k)(ab_flat, idx3, w3)
    blend_t = out3.transpose(1, 0, 2).reshape(12, n)

    mu_t, fr_t = _finalize(blend_t, means.T, quats.T, n)
    mu = mu_t.T
    fr = fr_t.T.reshape(n, 3, 3)
    return (mu, fr, scales, opacities, shs)


# speedup vs baseline: 14.2564x; 1.0302x over previous
import functools

import jax
import jax.numpy as jnp
from jax import lax
from jax.experimental import pallas as pl
from jax.experimental.pallas import tpu as pltpu
from jax.experimental.pallas import tpu_sc as plsc

SIGMA = 0.5
DQ_EPS = 1e-6

_NC = 2
_NS = 16
_LANES = 16
_NW = _NC * _NS

_BLK_N = 512
_BLK_D = 4096


def _q2r_rows(q):
    n = jnp.sqrt(q[0:1] * q[0:1] + q[1:2] * q[1:2]
                 + q[2:3] * q[2:3] + q[3:4] * q[3:4]) + 1e-8
    w = q[0:1] / n
    x = q[1:2] / n
    y = q[2:3] / n
    z = q[3:4] / n
    return [
        1 - 2 * (y * y + z * z), 2 * (x * y - w * z), 2 * (x * z + w * y),
        2 * (x * y + w * z), 1 - 2 * (x * x + z * z), 2 * (y * z - w * x),
        2 * (x * z - w * y), 2 * (y * z + w * x), 1 - 2 * (x * x + y * y),
    ]


def _node_table_body(qr_ref, qt_ref, xr_ref, xt_ref, out_ref, aug_ref):
    rr = _q2r_rows(qr_ref[...])
    rt = _q2r_rows(qt_ref[...])
    xr = xr_ref[...]
    xt = xt_ref[...]
    a = []
    for i in range(3):
        for j in range(3):
            a.append(rt[3 * i + 0] * rr[3 * j + 0]
                     + rt[3 * i + 1] * rr[3 * j + 1]
                     + rt[3 * i + 2] * rr[3 * j + 2])
    b = [xt[i:i + 1]
         - (a[3 * i + 0] * xr[0:1] + a[3 * i + 1] * xr[1:2]
            + a[3 * i + 2] * xr[2:3])
         for i in range(3)]
    zero = a[0] * 0.0
    out_ref[...] = jnp.concatenate(a + b + [zero] * 4, axis=0)
    cn = xr[0:1] * xr[0:1] + xr[1:2] * xr[1:2] + xr[2:3] * xr[2:3]
    aug_ref[...] = jnp.concatenate([xr, cn], axis=0)


def _node_table(qr_t, qt_t, xr_t, xt_t, m):
    return pl.pallas_call(
        _node_table_body,
        out_shape=[
            jax.ShapeDtypeStruct((16, m), jnp.float32),
            jax.ShapeDtypeStruct((4, m), jnp.float32),
        ],
    )(qr_t, qt_t, xr_t, xt_t)


def _topk_body(means_ref, nodes_ref, sw_ref, idx_ref, w_ref):
    means_blk = means_ref[...]
    nodes_aug = nodes_ref[...]
    bm = means_blk.shape[0]
    m = nodes_aug.shape[1]
    nodes3 = nodes_aug[0:3, :]
    cn = nodes_aug[3:4, :]
    mm = jnp.dot(means_blk[:, 0:3], nodes3,
                 preferred_element_type=jnp.float32)
    rn = (means_blk[:, 0:1] * means_blk[:, 0:1]
          + means_blk[:, 1:2] * means_blk[:, 1:2]
          + means_blk[:, 2:3] * means_blk[:, 2:3])
    rel = rn + cn - 2.0 * mm
    col = lax.broadcasted_iota(jnp.int32, (bm, m), 1)
    big = jnp.int32(0x7FFFFFFF)
    inf = jnp.float32(jnp.inf)
    vals, inds = [], []
    for k in range(8):
        mk = jnp.min(rel, axis=1, keepdims=True)
        ik = jnp.min(jnp.where(rel == mk, col, big),
                     axis=1, keepdims=True)
        vals.append(mk)
        inds.append(ik)
        if k < 7:
            rel = jnp.where(col == ik, inf, rel)
    d2k = jnp.concatenate(vals, axis=1)
    idx = jnp.minimum(jnp.concatenate(inds, axis=1), m - 1)
    w = jnp.exp(d2k * (-1.0 / (2.0 * SIGMA * SIGMA))) + sw_ref[...]
    w = jnp.maximum(w, 0.0)
    w = w / jnp.maximum(jnp.sum(w, axis=1, keepdims=True), DQ_EPS)
    idx_ref[...] = jnp.transpose(idx)[None]
    w_ref[...] = jnp.transpose(w)[None]


def _topk(means4, nodes_aug, sw, n, m, k):
    grid = n // _BLK_N
    p = n // _NW
    bpw = p // _BLK_N
    return pl.pallas_call(
        _topk_body,
        grid=(grid,),
        in_specs=[
            pl.BlockSpec((_BLK_N, 4), lambda i: (i, 0)),
            pl.BlockSpec((4, m), lambda i: (0, 0)),
            pl.BlockSpec((_BLK_N, k), lambda i: (i, 0)),
        ],
        out_specs=[
            pl.BlockSpec((1, k, _BLK_N), lambda i: (i // bpw, 0, i % bpw)),
            pl.BlockSpec((1, k, _BLK_N), lambda i: (i // bpw, 0, i % bpw)),
        ],
        out_shape=[
            jax.ShapeDtypeStruct((_NW, k, p), jnp.int32),
            jax.ShapeDtypeStruct((_NW, k, p), jnp.float32),
        ],
    )(means4, nodes_aug, sw)


def _sc_blend(n, m, k):
    p = n // _NW
    groups = p // _LANES
    mesh = plsc.VectorSubcoreMesh(core_axis_name="c", subcore_axis_name="s",
                                  num_cores=_NC, num_subcores=_NS)

    @functools.partial(
        pl.kernel,
        out_type=jax.ShapeDtypeStruct((_NW, 12, p), jnp.float32),
        mesh=mesh,
        compiler_params=pltpu.CompilerParams(needs_layout_passes=False),
        scratch_types=[
            pltpu.VMEM((m * 16,), jnp.float32),
            pltpu.VMEM((k, p), jnp.int32),
            pltpu.VMEM((k, p), jnp.float32),
            pltpu.VMEM((12, p), jnp.float32),
        ],
    )
    def blend(ab_hbm, idx_hbm, w_hbm, out_hbm, ab_v, idx_v, w_v, out_v):
        wid = lax.axis_index("s") * _NC + lax.axis_index("c")
        pltpu.sync_copy(ab_hbm, ab_v)
        pltpu.sync_copy(idx_hbm.at[wid], idx_v)
        pltpu.sync_copy(w_hbm.at[wid], w_v)

        def body(g, carry):
            col = g * _LANES
            acc = [jnp.zeros((_LANES,), jnp.float32) for _ in range(12)]
            for kk in range(k):
                iv = idx_v[kk, pl.ds(col, _LANES)] * 16
                wv = w_v[kk, pl.ds(col, _LANES)]
                for c in range(12):
                    acc[c] = acc[c] + wv * plsc.load_gather(ab_v, [iv + c])
            for c in range(12):
                out_v[c, pl.ds(col, _LANES)] = acc[c]
            return carry

        lax.fori_loop(0, groups, body, 0)
        pltpu.sync_copy(out_v, out_hbm.at[wid])

    return blend


def _final_body(blend_ref, means_ref, quats_ref, mu_ref, fr_ref):
    nw_blk = blend_ref.shape[0]
    rows = [jnp.concatenate([blend_ref[wi, c:c + 1, :]
                             for wi in range(nw_blk)], axis=1)
            for c in range(12)]
    f = rows[0:9]
    b = rows[9:12]
    mx = [means_ref[i:i + 1] for i in range(3)]
    rq = _q2r_rows(quats_ref[...])
    mu = [f[3 * i + 0] * mx[0] + f[3 * i + 1] * mx[1]
          + f[3 * i + 2] * mx[2] + b[i] for i in range(3)]
    fr = []
    for i in range(3):
        for j in range(3):
            fr.append(f[3 * i + 0] * rq[0 + j] + f[3 * i + 1] * rq[3 + j]
                      + f[3 * i + 2] * rq[6 + j])
    mu_ref[...] = jnp.concatenate(mu, axis=0)
    fr_ref[...] = jnp.concatenate(fr, axis=0)


def _finalize(out3, means_t, quats_t, n):
    grid = n // _BLK_D
    p = n // _NW
    wpb = _BLK_D // p
    return pl.pallas_call(
        _final_body,
        grid=(grid,),
        in_specs=[
            pl.BlockSpec((wpb, 12, p), lambda i: (i, 0, 0)),
            pl.BlockSpec((3, _BLK_D), lambda i: (0, i)),
            pl.BlockSpec((4, _BLK_D), lambda i: (0, i)),
        ],
        out_specs=[
            pl.BlockSpec((3, _BLK_D), lambda i: (0, i)),
            pl.BlockSpec((9, _BLK_D), lambda i: (0, i)),
        ],
        out_shape=[
            jax.ShapeDtypeStruct((3, n), jnp.float32),
            jax.ShapeDtypeStruct((9, n), jnp.float32),
        ],
    )(out3, means_t, quats_t)


def kernel(means, node_xyz_ref, node_quat_ref, node_xyz_t, node_quat_t,
           quats, skinning_weight, scales, opacities, shs, t):
    n = means.shape[0]
    m = node_xyz_ref.shape[0]
    k = skinning_weight.shape[1]

    ab16, nodes_aug = _node_table(node_quat_ref.T, node_quat_t.T,
                                  node_xyz_ref.T, node_xyz_t.T, m)
    ab_flat = ab16.T.reshape(m * 16)

    means4 = jnp.concatenate(
        [means, jnp.zeros((n, 1), jnp.float32)], axis=1)
    idx3, w3 = _topk(means4, nodes_aug, skinning_weight, n, m, k)

    out3 = _sc_blend(n, m, k)(ab_flat, idx3, w3)

    mu_t, fr_t = _finalize(out3, means.T, quats.T, n)
    mu = mu_t.T
    fr = fr_t.T.reshape(n, 3, 3)
    return (mu, fr, scales, opacities, shs)


# speedup vs baseline: 16.7286x; 1.1734x over previous
import functools

import jax
import jax.numpy as jnp
from jax import lax
from jax.experimental import pallas as pl
from jax.experimental.pallas import tpu as pltpu
from jax.experimental.pallas import tpu_sc as plsc

SIGMA = 0.5
DQ_EPS = 1e-6

_NC = 2
_NS = 16
_LANES = 16
_NW = _NC * _NS

_BLK_N = 512
_SPLIT = 2
_BLK_D = 4096


def _q2r_rows(q):
    n = jnp.sqrt(q[0:1] * q[0:1] + q[1:2] * q[1:2]
                 + q[2:3] * q[2:3] + q[3:4] * q[3:4]) + 1e-8
    w = q[0:1] / n
    x = q[1:2] / n
    y = q[2:3] / n
    z = q[3:4] / n
    return [
        1 - 2 * (y * y + z * z), 2 * (x * y - w * z), 2 * (x * z + w * y),
        2 * (x * y + w * z), 1 - 2 * (x * x + z * z), 2 * (y * z - w * x),
        2 * (x * z - w * y), 2 * (y * z + w * x), 1 - 2 * (x * x + y * y),
    ]


def _node_table_body(qr_ref, qt_ref, xr_ref, xt_ref, out_ref, aug_ref):
    rr = _q2r_rows(qr_ref[...])
    rt = _q2r_rows(qt_ref[...])
    xr = xr_ref[...]
    xt = xt_ref[...]
    a = []
    for i in range(3):
        for j in range(3):
            a.append(rt[3 * i + 0] * rr[3 * j + 0]
                     + rt[3 * i + 1] * rr[3 * j + 1]
                     + rt[3 * i + 2] * rr[3 * j + 2])
    b = [xt[i:i + 1]
         - (a[3 * i + 0] * xr[0:1] + a[3 * i + 1] * xr[1:2]
            + a[3 * i + 2] * xr[2:3])
         for i in range(3)]
    zero = a[0] * 0.0
    out_ref[...] = jnp.concatenate(a + b + [zero] * 4, axis=0)
    cn = xr[0:1] * xr[0:1] + xr[1:2] * xr[1:2] + xr[2:3] * xr[2:3]
    aug_ref[...] = jnp.concatenate([xr, cn], axis=0)


def _node_table(qr_t, qt_t, xr_t, xt_t, m):
    return pl.pallas_call(
        _node_table_body,
        out_shape=[
            jax.ShapeDtypeStruct((16, m), jnp.float32),
            jax.ShapeDtypeStruct((4, m), jnp.float32),
        ],
    )(qr_t, qt_t, xr_t, xt_t)


def _topk_body(means_ref, nodes_ref, sw_ref, idx_ref, w_ref):
    means_blk = means_ref[...]
    nodes_aug = nodes_ref[...]
    bm = means_blk.shape[0]
    m = nodes_aug.shape[1]
    nodes3 = nodes_aug[0:3, :]
    cn = nodes_aug[3:4, :]
    mm = jnp.dot(means_blk[:, 0:3], nodes3,
                 preferred_element_type=jnp.float32)
    rn = (means_blk[:, 0:1] * means_blk[:, 0:1]
          + means_blk[:, 1:2] * means_blk[:, 1:2]
          + means_blk[:, 2:3] * means_blk[:, 2:3])
    rel = rn + cn - 2.0 * mm
    col = lax.broadcasted_iota(jnp.int32, (bm, m), 1).astype(jnp.float32)
    inf = jnp.float32(jnp.inf)
    vals, inds = [], []
    for k in range(8):
        mk = jnp.min(rel, axis=1, keepdims=True)
        cand = jnp.where(rel == mk, col, inf)
        ik = jnp.min(cand, axis=1, keepdims=True)
        vals.append(mk)
        inds.append(ik)
        if k < 7:
            rel = jnp.where(cand == ik, inf, rel)
    d2k = jnp.concatenate(vals, axis=1)
    idx = jnp.minimum(jnp.concatenate(inds, axis=1),
                      jnp.float32(m - 1)).astype(jnp.int32)
    w = jnp.exp(d2k * (-1.0 / (2.0 * SIGMA * SIGMA))) + sw_ref[...]
    w = jnp.maximum(w, 0.0)
    w = w / jnp.maximum(jnp.sum(w, axis=1, keepdims=True), DQ_EPS)
    idx_ref[...] = jnp.transpose(idx)[None]
    w_ref[...] = jnp.transpose(w)[None]


def _topk(means4, nodes_aug, sw, n, m, k):
    grid = n // _BLK_N
    p = n // _NW
    bpw = p // _BLK_N
    return pl.pallas_call(
        _topk_body,
        grid=(grid,),
        in_specs=[
            pl.BlockSpec((_BLK_N, 4), lambda i: (i, 0)),
            pl.BlockSpec((4, m), lambda i: (0, 0)),
            pl.BlockSpec((_BLK_N, k), lambda i: (i, 0)),
        ],
        out_specs=[
            pl.BlockSpec((1, k, _BLK_N), lambda i: (i // bpw, 0, i % bpw)),
            pl.BlockSpec((1, k, _BLK_N), lambda i: (i // bpw, 0, i % bpw)),
        ],
        out_shape=[
            jax.ShapeDtypeStruct((_NW, k, p), jnp.int32),
            jax.ShapeDtypeStruct((_NW, k, p), jnp.float32),
        ],
    )(means4, nodes_aug, sw)


def _sc_blend(n, m, k):
    p = n // _NW
    groups = p // _LANES
    mesh = plsc.VectorSubcoreMesh(core_axis_name="c", subcore_axis_name="s",
                                  num_cores=_NC, num_subcores=_NS)

    @functools.partial(
        pl.kernel,
        out_type=jax.ShapeDtypeStruct((_NW, 12, p), jnp.float32),
        mesh=mesh,
        compiler_params=pltpu.CompilerParams(needs_layout_passes=False),
        scratch_types=[
            pltpu.VMEM((m * 16,), jnp.float32),
            pltpu.VMEM((k, p), jnp.int32),
            pltpu.VMEM((k, p), jnp.float32),
            pltpu.VMEM((12, p), jnp.float32),
        ],
    )
    def blend(ab_hbm, idx_hbm, w_hbm, out_hbm, ab_v, idx_v, w_v, out_v):
        wid = lax.axis_index("s") * _NC + lax.axis_index("c")
        pltpu.sync_copy(ab_hbm, ab_v)
        pltpu.sync_copy(idx_hbm.at[wid], idx_v)
        pltpu.sync_copy(w_hbm.at[wid], w_v)

        unroll = 4

        def body(g, carry):
            for u in range(unroll):
                col = (g * unroll + u) * _LANES
                acc = [jnp.zeros((_LANES,), jnp.float32) for _ in range(12)]
                for kk in range(k):
                    iv = idx_v[kk, pl.ds(col, _LANES)] * 16
                    wv = w_v[kk, pl.ds(col, _LANES)]
                    for c in range(12):
                        acc[c] = acc[c] + wv * plsc.load_gather(ab_v, [iv + c])
                for c in range(12):
                    out_v[c, pl.ds(col, _LANES)] = acc[c]
            return carry

        lax.fori_loop(0, groups // unroll, body, 0)
        pltpu.sync_copy(out_v, out_hbm.at[wid])

    return blend


def _final_body(blend_ref, means_ref, quats_ref, mu_ref, fr_ref):
    nw_blk = blend_ref.shape[0]
    rows = [jnp.concatenate([blend_ref[wi, c:c + 1, :]
                             for wi in range(nw_blk)], axis=1)
            for c in range(12)]
    f = rows[0:9]
    b = rows[9:12]
    mx = [means_ref[i:i + 1] for i in range(3)]
    rq = _q2r_rows(quats_ref[...])
    mu = [f[3 * i + 0] * mx[0] + f[3 * i + 1] * mx[1]
          + f[3 * i + 2] * mx[2] + b[i] for i in range(3)]
    fr = []
    for i in range(3):
        for j in range(3):
            fr.append(f[3 * i + 0] * rq[0 + j] + f[3 * i + 1] * rq[3 + j]
                      + f[3 * i + 2] * rq[6 + j])
    mu_ref[...] = jnp.concatenate(mu, axis=0)
    fr_ref[...] = jnp.concatenate(fr, axis=0)


def _finalize(out3, means_t, quats_t, n):
    grid = n // _BLK_D
    p = n // _NW
    wpb = _BLK_D // p
    return pl.pallas_call(
        _final_body,
        grid=(grid,),
        in_specs=[
            pl.BlockSpec((wpb, 12, p), lambda i: (i, 0, 0)),
            pl.BlockSpec((3, _BLK_D), lambda i: (0, i)),
            pl.BlockSpec((4, _BLK_D), lambda i: (0, i)),
        ],
        out_specs=[
            pl.BlockSpec((3, _BLK_D), lambda i: (0, i)),
            pl.BlockSpec((9, _BLK_D), lambda i: (0, i)),
        ],
        out_shape=[
            jax.ShapeDtypeStruct((3, n), jnp.float32),
            jax.ShapeDtypeStruct((9, n), jnp.float32),
        ],
    )(out3, means_t, quats_t)


def kernel(means, node_xyz_ref, node_quat_ref, node_xyz_t, node_quat_t,
           quats, skinning_weight, scales, opacities, shs, t):
    n = means.shape[0]
    m = node_xyz_ref.shape[0]
    k = skinning_weight.shape[1]

    ab16, nodes_aug = _node_table(node_quat_ref.T, node_quat_t.T,
                                  node_xyz_ref.T, node_xyz_t.T, m)
    ab_flat = ab16.T.reshape(m * 16)

    means4 = jnp.concatenate(
        [means, jnp.zeros((n, 1), jnp.float32)], axis=1)
    nh = n // _SPLIT
    blend = _sc_blend(nh, m, k)
    mus, frs = [], []
    for h in range(_SPLIT):
        sl = slice(h * nh, (h + 1) * nh)
        idx3, w3 = _topk(means4[sl], nodes_aug, skinning_weight[sl],
                         nh, m, k)
        out3 = blend(ab_flat, idx3, w3)
        mu_t, fr_t = _finalize(out3, means[sl].T, quats[sl].T, nh)
        mus.append(mu_t)
        frs.append(fr_t)
    mu_t = jnp.concatenate(mus, axis=1)
    fr_t = jnp.concatenate(frs, axis=1)
    mu = mu_t.T
    fr = fr_t.T.reshape(n, 3, 3)
    return (mu, fr, scales, opacities, shs)
